# Initial kernel scaffold; baseline (speedup 1.0000x reference)
import jax
import jax.numpy as jnp
from jax.experimental import pallas as pl


def kernel(x, adj, mask1, mask2, w1, w2):
    raise NotImplementedError("write your pallas kernel here")


# speedup vs baseline: 1.1120x; 1.1120x over previous
import functools

import jax
import jax.numpy as jnp
from jax import lax
from jax.experimental import pallas as pl
from jax.experimental.pallas import tpu as pltpu


def _vmem_limit(block_bytes: int) -> int:
    need = 2 * block_bytes + (4 << 20)
    return int(min(max(need, 16 << 20), 64 << 20))


def _prep_kernel(adj_ref, m1_ref, m2_ref, a_ref, d_ref):
    a = adj_ref[...] * m1_ref[...] * m2_ref[...]
    rs = jnp.sum(a, axis=1, keepdims=True) + 1.0
    d_ref[...] = jnp.where(rs > 0.0, lax.rsqrt(rs), 0.0)
    a_ref[...] = a.astype(a_ref.dtype)


def _layer_kernel(a_ref, h_ref, d_ref, w_ref, o_ref, *, tile, relu):
    j = pl.program_id(0)
    y = (h_ref[...].astype(jnp.float32) * d_ref[...]).astype(jnp.bfloat16)
    a_blk = a_ref[...].astype(jnp.bfloat16)
    z = lax.dot_general(y, a_blk, (((1,), (0,)), ((), ())),
                        preferred_element_type=jnp.float32)
    d_j = d_ref[:, pl.ds(j * tile, tile)]
    h_j = h_ref[:, pl.ds(j * tile, tile)].astype(jnp.float32)
    hc = d_j * (z + d_j * h_j)
    g = lax.dot_general(w_ref[...], hc, (((1,), (0,)), ((), ())),
                        preferred_element_type=jnp.float32)
    if relu:
        g = jnp.maximum(g, 0.0)
    o_ref[...] = g.astype(o_ref.dtype)


def kernel(x, adj, mask1, mask2, w1, w2):
    n = adj.shape[0]
    tile = 256
    assert n % tile == 0, "node count must be a multiple of 256"
    num_tiles = n // tile
    f32 = jnp.float32
    a_store = jnp.bfloat16
    a_isz = jnp.dtype(a_store).itemsize

    prep_bytes = 3 * tile * n * 4 + tile * n * a_isz + tile * 4
    a_m, d_col = pl.pallas_call(
        _prep_kernel,
        grid=(num_tiles,),
        in_specs=[pl.BlockSpec((tile, n), lambda i: (i, 0))] * 3,
        out_specs=[pl.BlockSpec((tile, n), lambda i: (i, 0)),
                   pl.BlockSpec((tile, 1), lambda i: (i, 0))],
        out_shape=[jax.ShapeDtypeStruct((n, n), a_store),
                   jax.ShapeDtypeStruct((n, 1), f32)],
        compiler_params=pltpu.CompilerParams(
            dimension_semantics=("parallel",),
            vmem_limit_bytes=_vmem_limit(prep_bytes)),
    )(adj.astype(f32), mask1.astype(f32), mask2.astype(f32))
    d_row = d_col.reshape(1, n)

    h = x.astype(f32).T
    for ln, w in enumerate((w1, w2)):
        f_out, f_in = w.shape
        layer_bytes = (n * tile * a_isz + f_in * n * 4 + n * 4
                       + f_out * f_in * 4 + f_out * tile * 4)
        fn = functools.partial(_layer_kernel, tile=tile, relu=(ln == 0))
        h = pl.pallas_call(
            fn,
            grid=(num_tiles,),
            in_specs=[
                pl.BlockSpec((n, tile), lambda j: (0, j)),
                pl.BlockSpec((f_in, n), lambda j: (0, 0)),
                pl.BlockSpec((1, n), lambda j: (0, 0)),
                pl.BlockSpec((f_out, f_in), lambda j: (0, 0)),
            ],
            out_specs=pl.BlockSpec((f_out, tile), lambda j: (0, j)),
            out_shape=jax.ShapeDtypeStruct((f_out, n), f32),
            compiler_params=pltpu.CompilerParams(
                dimension_semantics=("parallel",),
                vmem_limit_bytes=_vmem_limit(layer_bytes)),
        )(a_m, h, d_row, w.astype(f32))

    return h.T
